# baseline (speedup 1.0000x reference)
import functools

import jax
import jax.numpy as jnp
from jax import lax
from jax.experimental import pallas as pl
from jax.experimental.pallas import tpu as pltpu
from jax.experimental.pallas import tpu_sc as plsc

_INFO = plsc.get_sparse_core_info()
_NC = _INFO.num_cores
_NS = _INFO.num_subcores
_NW = _NC * _NS
_L = _INFO.num_lanes


@functools.cache
def _make_gather(n_pos: int, dim: int):
    assert n_pos % _NW == 0
    b_per_w = n_pos // _NW
    assert b_per_w % 8 == 0 and b_per_w % _L == 0
    half = dim // 2

    mesh = plsc.VectorSubcoreMesh(core_axis_name="c", subcore_axis_name="s")

    @functools.partial(
        pl.kernel,
        mesh=mesh,
        compiler_params=pltpu.CompilerParams(use_tc_tiling_on_sc=False),
        out_type=(
            jax.ShapeDtypeStruct((n_pos, 2, half), jnp.float32),
            jax.ShapeDtypeStruct((n_pos, 2, half), jnp.float32),
        ),
        scratch_types=[
            pltpu.VMEM((b_per_w,), jnp.int32),
            pltpu.VMEM((b_per_w, half), jnp.float32),
            pltpu.VMEM((b_per_w, half), jnp.float32),
            pltpu.SemaphoreType.DMA,
            pltpu.SemaphoreType.DMA,
            pltpu.SemaphoreType.DMA,
            pltpu.SemaphoreType.DMA,
        ],
    )
    def gather(cos_hbm, sin_hbm, idx_hbm, cos_out, sin_out,
               idx_v, cos_v, sin_v, sem_c, sem_s, sem_wc, sem_ws):
        wid = lax.axis_index("s") * _NC + lax.axis_index("c")
        base = wid * b_per_w
        pltpu.sync_copy(idx_hbm.at[pl.ds(base, b_per_w)], idx_v)
        for i in range(b_per_w // _L):
            sl = pl.ds(i * _L, _L)
            idx_v[sl] = idx_v[sl] * 2
        cp_c = pltpu.async_copy(cos_hbm.at[idx_v], cos_v, sem_c)
        cp_s = pltpu.async_copy(sin_hbm.at[idx_v], sin_v, sem_s)
        out_sl = pl.ds(base, b_per_w)
        cp_c.wait()
        wc0 = pltpu.async_copy(cos_v, cos_out.at[out_sl, 0], sem_wc)
        wc1 = pltpu.async_copy(cos_v, cos_out.at[out_sl, 1], sem_wc)
        cp_s.wait()
        ws0 = pltpu.async_copy(sin_v, sin_out.at[out_sl, 0], sem_ws)
        ws1 = pltpu.async_copy(sin_v, sin_out.at[out_sl, 1], sem_ws)
        wc0.wait()
        wc1.wait()
        ws0.wait()
        ws1.wait()

    return gather


def kernel(x, position_ids, cached_cos, cached_sin):
    del x
    max_pos, dim = cached_cos.shape[-2], cached_cos.shape[-1]
    n_pos = position_ids.shape[0]
    cos_tab = cached_cos.reshape(2 * max_pos, dim // 2)
    sin_tab = cached_sin.reshape(2 * max_pos, dim // 2)
    cos, sin = _make_gather(n_pos, dim)(cos_tab, sin_tab, position_ids)
    return (cos.reshape(1, 1, n_pos, dim), sin.reshape(1, 1, n_pos, dim))


# speedup vs baseline: 2.1208x; 2.1208x over previous
import functools

import jax
import jax.numpy as jnp
from jax import lax
from jax.experimental import pallas as pl
from jax.experimental.pallas import tpu as pltpu
from jax.experimental.pallas import tpu_sc as plsc

_INFO = plsc.get_sparse_core_info()
_NC = _INFO.num_cores
_NS = _INFO.num_subcores
_NW = _NC * _NS


@functools.cache
def _make_gather(n_pos: int, dim: int):
    assert n_pos % _NW == 0
    b_per_w = n_pos // _NW
    assert b_per_w % 8 == 0

    mesh = plsc.VectorSubcoreMesh(core_axis_name="c", subcore_axis_name="s")

    @functools.partial(
        pl.kernel,
        mesh=mesh,
        compiler_params=pltpu.CompilerParams(skip_device_barrier=True),
        out_type=(
            jax.ShapeDtypeStruct((n_pos, dim), jnp.float32),
            jax.ShapeDtypeStruct((n_pos, dim), jnp.float32),
        ),
        scratch_types=[
            pltpu.VMEM((b_per_w,), jnp.int32),
            pltpu.VMEM((b_per_w, dim), jnp.float32),
            pltpu.VMEM((b_per_w, dim), jnp.float32),
            pltpu.SemaphoreType.DMA,
            pltpu.SemaphoreType.DMA,
            pltpu.SemaphoreType.DMA,
            pltpu.SemaphoreType.DMA,
        ],
    )
    def gather(cos_hbm, sin_hbm, idx_hbm, cos_out, sin_out,
               idx_v, cos_v, sin_v, sem_c, sem_s, sem_wc, sem_ws):
        wid = lax.axis_index("s") * _NC + lax.axis_index("c")
        base = wid * b_per_w
        pltpu.sync_copy(idx_hbm.at[pl.ds(base, b_per_w)], idx_v)
        cp_c = pltpu.async_copy(cos_hbm.at[idx_v], cos_v, sem_c)
        cp_s = pltpu.async_copy(sin_hbm.at[idx_v], sin_v, sem_s)
        cp_c.wait()
        wr_c = pltpu.async_copy(cos_v, cos_out.at[pl.ds(base, b_per_w)], sem_wc)
        cp_s.wait()
        wr_s = pltpu.async_copy(sin_v, sin_out.at[pl.ds(base, b_per_w)], sem_ws)
        wr_c.wait()
        wr_s.wait()

    return gather


def kernel(x, position_ids, cached_cos, cached_sin):
    del x
    max_pos, dim = cached_cos.shape[-2], cached_cos.shape[-1]
    n_pos = position_ids.shape[0]
    cos_tab = cached_cos.reshape(max_pos, dim)
    sin_tab = cached_sin.reshape(max_pos, dim)
    cos, sin = _make_gather(n_pos, dim)(cos_tab, sin_tab, position_ids)
    return (cos.reshape(1, 1, n_pos, dim), sin.reshape(1, 1, n_pos, dim))
